# baseline (speedup 1.0000x reference)
import functools

import jax
import jax.numpy as jnp
from jax import lax
from jax.experimental import pallas as pl
from jax.experimental.pallas import tpu as pltpu
from jax.experimental.pallas import tpu_sc as plsc

V = 1000000
D = 64
B = 16384
NC = 2
NS = 16
L = 16
NW = NC * NS
BPW = B // NW
NG = BPW // L


def _glove_body(cw_hbm, tw_hbm, co_hbm, wt_hbm, wc_hbm, wo_hbm, bv_hbm,
                bu_hbm, out_hbm, cidx_v, tidx_v, ce_v, te_v, bv_v, bu_v,
                co_v, wt_v, tp_v, accv_v, red_v, outrow_v, shared_v,
                sem1, sem2, sem3, sem4):
    c = lax.axis_index("c")
    s = lax.axis_index("s")
    wid = c * NS + s
    base = wid * BPW

    pltpu.sync_copy(cw_hbm.at[pl.ds(base, BPW)], cidx_v)
    pltpu.sync_copy(tw_hbm.at[pl.ds(base, BPW)], tidx_v)
    pltpu.sync_copy(co_hbm.at[pl.ds(base, BPW)], co_v)
    pltpu.sync_copy(wt_hbm.at[pl.ds(base, BPW)], wt_v)

    cp1 = pltpu.async_copy(wc_hbm.at[cidx_v], ce_v, sem1)
    cp2 = pltpu.async_copy(wo_hbm.at[tidx_v], te_v, sem2)
    cp3 = pltpu.async_copy(bv_hbm.at[cidx_v], bv_v, sem3)
    cp4 = pltpu.async_copy(bu_hbm.at[tidx_v], bu_v, sem4)
    cp1.wait()
    cp2.wait()
    cp3.wait()
    cp4.wait()

    iota16 = lax.iota(jnp.int32, L)

    def group_body(g, acc):
        gb = g * L
        for j in range(L):
            p = ce_v[gb + j, pl.ds(0, L)] * te_v[gb + j, pl.ds(0, L)]
            for k in range(1, D // L):
                p += (ce_v[gb + j, pl.ds(k * L, L)]
                      * te_v[gb + j, pl.ds(k * L, L)])
            tp_v[j, pl.ds(0, L)] = p
        dots = plsc.load_gather(tp_v, [iota16, jnp.zeros((L,), jnp.int32)])
        for col in range(1, L):
            dots += plsc.load_gather(
                tp_v, [iota16, jnp.full((L,), col, jnp.int32)])
        sv = dots + bv_v[pl.ds(gb, L)] + bu_v[pl.ds(gb, L)] - co_v[pl.ds(gb, L)]
        return acc + wt_v[pl.ds(gb, L)] * sv * sv

    acc = lax.fori_loop(0, NG, group_body, jnp.zeros((L,), jnp.float32))

    accv_v[...] = acc
    pltpu.sync_copy(accv_v, shared_v.at[s])
    plsc.subcore_barrier()

    @pl.when(s == 0)
    def _():
        pltpu.sync_copy(shared_v, red_v)
        t16 = red_v[0, pl.ds(0, L)]
        for i in range(1, NS):
            t16 += red_v[i, pl.ds(0, L)]
        total = jnp.sum(t16)
        outrow_v[...] = jnp.full((L,), total, jnp.float32)
        pltpu.sync_copy(outrow_v, out_hbm.at[c])


@jax.jit
def _glove(cw, tw, co, wt, wc, wo, bv, bu):
    mesh = plsc.VectorSubcoreMesh(core_axis_name="c", subcore_axis_name="s",
                                  num_cores=NC, num_subcores=NS)
    f = pl.kernel(
        _glove_body,
        out_type=jax.ShapeDtypeStruct((NC, L), jnp.float32),
        mesh=mesh,
        compiler_params=pltpu.CompilerParams(needs_layout_passes=False,
                                             use_tc_tiling_on_sc=False),
        scratch_types=[
            pltpu.VMEM((BPW,), jnp.int32),
            pltpu.VMEM((BPW,), jnp.int32),
            pltpu.VMEM((BPW, D), jnp.float32),
            pltpu.VMEM((BPW, D), jnp.float32),
            pltpu.VMEM((BPW,), jnp.float32),
            pltpu.VMEM((BPW,), jnp.float32),
            pltpu.VMEM((BPW,), jnp.float32),
            pltpu.VMEM((BPW,), jnp.float32),
            pltpu.VMEM((L, L + 1), jnp.float32),
            pltpu.VMEM((L,), jnp.float32),
            pltpu.VMEM((NS, L), jnp.float32),
            pltpu.VMEM((L,), jnp.float32),
            pltpu.VMEM_SHARED((NS, L), jnp.float32),
            pltpu.SemaphoreType.DMA,
            pltpu.SemaphoreType.DMA,
            pltpu.SemaphoreType.DMA,
            pltpu.SemaphoreType.DMA,
        ],
    )
    return f(cw, tw, co, wt, wc, wo, bv, bu)


def kernel(center_words, target_words, coocs, weighting, W_center, W_outside,
           b_v, b_u):
    cw = center_words.reshape(B).astype(jnp.int32)
    tw = target_words.reshape(B).astype(jnp.int32)
    co = coocs.reshape(B)
    wt = weighting.reshape(B)
    bv = b_v.reshape(V)
    bu = b_u.reshape(V)
    out = _glove(cw, tw, co, wt, W_center, W_outside, bv, bu)
    return out[0, 0] + out[1, 0]
